# baseline (speedup 1.0000x reference)
import functools

import jax
import jax.numpy as jnp
from jax import lax
from jax.experimental import pallas as pl
from jax.experimental.pallas import tpu as pltpu
from jax.experimental.pallas import tpu_sc as plsc

N = 1024
D = 64
NW = 32
E = N * N
E_PER_W = E // NW
IDX_W = 128
CHUNK = 512
SUB = CHUNK // IDX_W
CHUNKS = E_PER_W // CHUNK
NODES_PER_W = N // NW

_mesh = plsc.VectorSubcoreMesh(core_axis_name="c", subcore_axis_name="s")


@functools.partial(
    pl.kernel,
    out_type=(
        jax.ShapeDtypeStruct((N, D), jnp.float32),
        jax.ShapeDtypeStruct((E, D), jnp.float32),
        jax.ShapeDtypeStruct((8, D), jnp.float32),
    ),
    mesh=_mesh,
    scratch_types=[
        pltpu.VMEM((SUB, IDX_W), jnp.int32),
        pltpu.VMEM((CHUNK, D), jnp.float32),
        pltpu.VMEM((NODES_PER_W,), jnp.int32),
        pltpu.VMEM((NODES_PER_W, D), jnp.float32),
        pltpu.VMEM((8,), jnp.int32),
        pltpu.VMEM((8, D), jnp.float32),
        pltpu.SemaphoreType.DMA,
    ],
    compiler_params=pltpu.CompilerParams(use_tc_tiling_on_sc=False),
)
def _sc_embed(nodes_hbm, edges_hbm, tsteps_hbm, ntab_hbm, etab_hbm, ttab_hbm,
              node_out, edge_out, time_out,
              eidx_v, erows_v, nidx_v, nrows_v, tidx_v, trows_v, sem):
    wid = lax.axis_index("s") * 2 + lax.axis_index("c")

    nbase = pl.multiple_of(wid * NODES_PER_W, NODES_PER_W)
    pltpu.sync_copy(nodes_hbm.at[pl.ds(nbase, NODES_PER_W)], nidx_v)
    pltpu.async_copy(ntab_hbm.at[nidx_v], nrows_v, sem).wait()
    pltpu.sync_copy(nrows_v, node_out.at[pl.ds(nbase, NODES_PER_W)])

    @pl.when(wid == 0)
    def _():
        pltpu.sync_copy(tsteps_hbm, tidx_v)
        pltpu.async_copy(ttab_hbm.at[tidx_v], trows_v, sem).wait()
        pltpu.sync_copy(trows_v, time_out)

    def body(i, carry):
        base = pl.multiple_of(wid * E_PER_W + i * CHUNK, CHUNK)
        row = pl.multiple_of(wid * (E_PER_W // IDX_W) + i * SUB, SUB)
        pltpu.sync_copy(edges_hbm.at[pl.ds(row, SUB)], eidx_v)
        copies = [
            pltpu.async_copy(
                etab_hbm.at[eidx_v.at[j]],
                erows_v.at[pl.ds(j * IDX_W, IDX_W)],
                sem,
            )
            for j in range(SUB)
        ]
        for c in copies:
            c.wait()
        pltpu.sync_copy(erows_v, edge_out.at[pl.ds(base, CHUNK)])
        return carry

    lax.fori_loop(0, CHUNKS, body, 0)


def kernel(nodes, edges, timestep, node_table, edge_table, time_table):
    tsteps = jnp.full((8,), timestep, dtype=jnp.int32)
    node_emb, edge_flat, time_rows = _sc_embed(
        nodes.astype(jnp.int32),
        edges.reshape(E // IDX_W, IDX_W).astype(jnp.int32),
        tsteps,
        node_table,
        edge_table,
        time_table,
    )
    return node_emb, edge_flat.reshape(N, N, D), time_rows[0]


# speedup vs baseline: 2.0261x; 2.0261x over previous
import functools

import jax
import jax.numpy as jnp
from jax import lax
from jax.experimental import pallas as pl
from jax.experimental.pallas import tpu as pltpu
from jax.experimental.pallas import tpu_sc as plsc

N = 1024
D = 64
NW = 32
E = N * N
E_PER_W = E // NW
CHUNK = 512
CHUNKS = E_PER_W // CHUNK
GROUPS = CHUNK // 16
NODES_PER_W = N // NW

_mesh = plsc.VectorSubcoreMesh(core_axis_name="c", subcore_axis_name="s")


@functools.partial(
    pl.kernel,
    out_type=(
        jax.ShapeDtypeStruct((N, D), jnp.float32),
        jax.ShapeDtypeStruct((E * D,), jnp.float32),
        jax.ShapeDtypeStruct((8, D), jnp.float32),
    ),
    mesh=_mesh,
    scratch_types=[
        pltpu.VMEM((D * 8,), jnp.float32),
        pltpu.VMEM((2, CHUNK), jnp.int32),
        pltpu.VMEM((2, CHUNK * D), jnp.float32),
        pltpu.VMEM((NODES_PER_W,), jnp.int32),
        pltpu.VMEM((NODES_PER_W, D), jnp.float32),
        pltpu.VMEM((8,), jnp.int32),
        pltpu.VMEM((8, D), jnp.float32),
        pltpu.SemaphoreType.DMA,
        pltpu.SemaphoreType.DMA,
        pltpu.SemaphoreType.DMA,
        pltpu.SemaphoreType.DMA,
        pltpu.SemaphoreType.DMA,
    ],
    compiler_params=pltpu.CompilerParams(
        use_tc_tiling_on_sc=False, needs_layout_passes=False),
)
def _sc_embed(nodes_hbm, edges_hbm, tsteps_hbm, ntab_hbm, etab_hbm, ttab_hbm,
              node_out, edge_out, time_out,
              etab_v, eidx_v, erows_v, nidx_v, nrows_v, tidx_v, trows_v,
              sem_i0, sem_i1, sem_o0, sem_o1, sem_m):
    wid = lax.axis_index("s") * 2 + lax.axis_index("c")
    ebase = wid * E_PER_W
    sem_i = (sem_i0, sem_i1)
    sem_o = (sem_o0, sem_o1)

    for par in range(2):
        pltpu.async_copy(
            edges_hbm.at[pl.ds(pl.multiple_of(ebase + par * CHUNK, CHUNK), CHUNK)],
            eidx_v.at[par], sem_i[par])
    pltpu.sync_copy(etab_hbm, etab_v)

    nbase = pl.multiple_of(wid * NODES_PER_W, NODES_PER_W)
    pltpu.sync_copy(nodes_hbm.at[pl.ds(nbase, NODES_PER_W)], nidx_v)
    pltpu.async_copy(ntab_hbm.at[nidx_v], nrows_v, sem_m).wait()
    pltpu.sync_copy(nrows_v, node_out.at[pl.ds(nbase, NODES_PER_W)])

    @pl.when(wid == 0)
    def _():
        pltpu.sync_copy(tsteps_hbm, tidx_v)
        pltpu.async_copy(ttab_hbm.at[tidx_v], trows_v, sem_m).wait()
        pltpu.sync_copy(trows_v, time_out)

    iota16 = lax.iota(jnp.int32, 16)
    iota64 = iota16 * D

    def compute_chunk(idx_ref, out_ref):
        def group(g, carry):
            e_vec = idx_ref[pl.ds(g * 16, 16)]
            src = e_vec * D
            dst = g * (16 * D) + iota64
            for d in range(D):
                vals = plsc.load_gather(etab_v, [src + d])
                plsc.store_scatter(out_ref, [dst + d], vals)
            return carry
        lax.fori_loop(0, GROUPS, group, 0)

    def body(p, carry):
        for par in range(2):
            c = p * 2 + par
            hbm_off = pl.multiple_of((ebase + c * CHUNK) * D, CHUNK * D)
            idx_off = pl.multiple_of(ebase + (c + 2) * CHUNK, CHUNK)
            pltpu.make_async_copy(
                edges_hbm.at[pl.ds(0, CHUNK)], eidx_v.at[par], sem_i[par]
            ).wait()
            @pl.when(p > 0)
            def _():
                pltpu.make_async_copy(
                    erows_v.at[par], edge_out.at[pl.ds(0, CHUNK * D)], sem_o[par]
                ).wait()
            compute_chunk(eidx_v.at[par], erows_v.at[par])
            @pl.when(p < (CHUNKS // 2) - 1)
            def _():
                pltpu.async_copy(
                    edges_hbm.at[pl.ds(idx_off, CHUNK)], eidx_v.at[par],
                    sem_i[par])
            pltpu.async_copy(
                erows_v.at[par], edge_out.at[pl.ds(hbm_off, CHUNK * D)],
                sem_o[par])
        return carry

    lax.fori_loop(0, CHUNKS // 2, body, 0)
    for par in range(2):
        pltpu.make_async_copy(
            erows_v.at[par], edge_out.at[pl.ds(0, CHUNK * D)], sem_o[par]
        ).wait()


def kernel(nodes, edges, timestep, node_table, edge_table, time_table):
    tsteps = jnp.full((8,), timestep, dtype=jnp.int32)
    node_emb, edge_flat, time_rows = _sc_embed(
        nodes.astype(jnp.int32),
        edges.reshape(E).astype(jnp.int32),
        tsteps,
        node_table,
        edge_table.reshape(D * 8),
        time_table,
    )
    return node_emb, edge_flat.reshape(N, N, D), time_rows[0]


# speedup vs baseline: 2.3380x; 1.1539x over previous
import functools

import jax
import jax.numpy as jnp
from jax import lax
from jax.experimental import pallas as pl
from jax.experimental.pallas import tpu as pltpu
from jax.experimental.pallas import tpu_sc as plsc

N = 1024
D = 64
NW = 32
E = N * N
E_PER_W = E // NW
CHUNK = 512
CHUNKS = E_PER_W // CHUNK
GROUPS = CHUNK // 16
NODES_PER_W = N // NW

_mesh = plsc.VectorSubcoreMesh(core_axis_name="c", subcore_axis_name="s")


@functools.partial(
    pl.kernel,
    out_type=(
        jax.ShapeDtypeStruct((N, D), jnp.float32),
        jax.ShapeDtypeStruct((E * D,), jnp.float32),
        jax.ShapeDtypeStruct((8, D), jnp.float32),
    ),
    mesh=_mesh,
    scratch_types=[
        pltpu.VMEM((D * 8,), jnp.float32),
        pltpu.VMEM((2, CHUNK), jnp.int32),
        pltpu.VMEM((2, CHUNK * D), jnp.float32),
        pltpu.VMEM((NODES_PER_W,), jnp.int32),
        pltpu.VMEM((NODES_PER_W, D), jnp.float32),
        pltpu.VMEM((8,), jnp.int32),
        pltpu.VMEM((8, D), jnp.float32),
        pltpu.SemaphoreType.DMA,
        pltpu.SemaphoreType.DMA,
        pltpu.SemaphoreType.DMA,
        pltpu.SemaphoreType.DMA,
        pltpu.SemaphoreType.DMA,
    ],
    compiler_params=pltpu.CompilerParams(
        use_tc_tiling_on_sc=False, needs_layout_passes=False),
)
def _sc_embed(nodes_hbm, edges_hbm, tsteps_hbm, ntab_hbm, etab_hbm, ttab_hbm,
              node_out, edge_out, time_out,
              etab_v, eidx_v, erows_v, nidx_v, nrows_v, tidx_v, trows_v,
              sem_i0, sem_i1, sem_o0, sem_o1, sem_m):
    wid = lax.axis_index("s") * 2 + lax.axis_index("c")
    ebase = wid * E_PER_W
    sem_i = (sem_i0, sem_i1)
    sem_o = (sem_o0, sem_o1)

    for par in range(2):
        pltpu.async_copy(
            edges_hbm.at[pl.ds(pl.multiple_of(ebase + par * CHUNK, CHUNK), CHUNK)],
            eidx_v.at[par], sem_i[par])
    pltpu.sync_copy(etab_hbm, etab_v)

    nbase = pl.multiple_of(wid * NODES_PER_W, NODES_PER_W)
    pltpu.sync_copy(nodes_hbm.at[pl.ds(nbase, NODES_PER_W)], nidx_v)
    pltpu.async_copy(ntab_hbm.at[nidx_v], nrows_v, sem_m).wait()
    pltpu.sync_copy(nrows_v, node_out.at[pl.ds(nbase, NODES_PER_W)])

    @pl.when(wid == 0)
    def _():
        pltpu.sync_copy(tsteps_hbm, tidx_v)
        pltpu.async_copy(ttab_hbm.at[tidx_v], trows_v, sem_m).wait()
        pltpu.sync_copy(trows_v, time_out)

    iota16 = lax.iota(jnp.int32, 16)
    iota64 = iota16 * D

    def compute_chunk(idx_ref, out_ref):
        @plsc.parallel_loop(0, GROUPS, step=1, unroll=2)
        def group(g):
            e_vec = idx_ref[pl.ds(g * 16, 16)]
            src = e_vec * D
            dst = g * (16 * D) + iota64
            for d in range(D):
                vals = plsc.load_gather(etab_v, [src + d])
                plsc.store_scatter(out_ref, [dst + d], vals)

    def body(p, carry):
        for par in range(2):
            c = p * 2 + par
            hbm_off = pl.multiple_of((ebase + c * CHUNK) * D, CHUNK * D)
            idx_off = pl.multiple_of(ebase + (c + 2) * CHUNK, CHUNK)
            pltpu.make_async_copy(
                edges_hbm.at[pl.ds(0, CHUNK)], eidx_v.at[par], sem_i[par]
            ).wait()
            @pl.when(p > 0)
            def _():
                pltpu.make_async_copy(
                    erows_v.at[par], edge_out.at[pl.ds(0, CHUNK * D)], sem_o[par]
                ).wait()
            compute_chunk(eidx_v.at[par], erows_v.at[par])
            @pl.when(p < (CHUNKS // 2) - 1)
            def _():
                pltpu.async_copy(
                    edges_hbm.at[pl.ds(idx_off, CHUNK)], eidx_v.at[par],
                    sem_i[par])
            pltpu.async_copy(
                erows_v.at[par], edge_out.at[pl.ds(hbm_off, CHUNK * D)],
                sem_o[par])
        return carry

    lax.fori_loop(0, CHUNKS // 2, body, 0)
    for par in range(2):
        pltpu.make_async_copy(
            erows_v.at[par], edge_out.at[pl.ds(0, CHUNK * D)], sem_o[par]
        ).wait()


def kernel(nodes, edges, timestep, node_table, edge_table, time_table):
    tsteps = jnp.full((8,), timestep, dtype=jnp.int32)
    node_emb, edge_flat, time_rows = _sc_embed(
        nodes.astype(jnp.int32),
        edges.reshape(E).astype(jnp.int32),
        tsteps,
        node_table,
        edge_table.reshape(D * 8),
        time_table,
    )
    return node_emb, edge_flat.reshape(N, N, D), time_rows[0]


# speedup vs baseline: 5.8607x; 2.5067x over previous
import functools

import jax
import jax.numpy as jnp
from jax import lax
from jax.experimental import pallas as pl
from jax.experimental.pallas import tpu as pltpu
from jax.experimental.pallas import tpu_sc as plsc

N = 1024
D = 64
NW = 32
E = N * N
E_PER_W = E // NW
CHUNK = 512
PADW = D + 1
CHUNKS = E_PER_W // CHUNK
GROUPS = CHUNK // 16
NODES_PER_W = N // NW

_mesh = plsc.VectorSubcoreMesh(core_axis_name="c", subcore_axis_name="s")

_GATHER_DNUMS = lax.GatherDimensionNumbers(
    offset_dims=(), collapsed_slice_dims=(0,), start_index_map=(0,))


def _vperm(tab_d, e_vec):
    return lax.gather(
        tab_d, e_vec[:, None], _GATHER_DNUMS, slice_sizes=(1,),
        mode=lax.GatherScatterMode.PROMISE_IN_BOUNDS)


@functools.partial(
    pl.kernel,
    out_type=(
        jax.ShapeDtypeStruct((N, D), jnp.float32),
        jax.ShapeDtypeStruct((E, D), jnp.float32),
        jax.ShapeDtypeStruct((8, D), jnp.float32),
    ),
    mesh=_mesh,
    scratch_types=[
        pltpu.VMEM((D * 16,), jnp.float32),
        pltpu.VMEM((2, CHUNK), jnp.int32),
        pltpu.VMEM((2, CHUNK, PADW), jnp.float32),
        pltpu.VMEM((NODES_PER_W,), jnp.int32),
        pltpu.VMEM((NODES_PER_W, D), jnp.float32),
        pltpu.VMEM((8,), jnp.int32),
        pltpu.VMEM((8, D), jnp.float32),
        pltpu.SemaphoreType.DMA,
        pltpu.SemaphoreType.DMA,
        pltpu.SemaphoreType.DMA,
        pltpu.SemaphoreType.DMA,
        pltpu.SemaphoreType.DMA,
    ],
    compiler_params=pltpu.CompilerParams(
        use_tc_tiling_on_sc=False, needs_layout_passes=False),
)
def _sc_embed(nodes_hbm, edges_hbm, tsteps_hbm, ntab_hbm, etabt_hbm, ttab_hbm,
              node_out, edge_out, time_out,
              etab_v, eidx_v, erows_v, nidx_v, nrows_v, tidx_v, trows_v,
              sem_i0, sem_i1, sem_o0, sem_o1, sem_m):
    wid = lax.axis_index("s") * 2 + lax.axis_index("c")
    ebase = wid * E_PER_W
    sem_i = (sem_i0, sem_i1)
    sem_o = (sem_o0, sem_o1)

    for par in range(2):
        pltpu.async_copy(
            edges_hbm.at[pl.ds(pl.multiple_of(ebase + par * CHUNK, CHUNK), CHUNK)],
            eidx_v.at[par], sem_i[par])
    pltpu.sync_copy(etabt_hbm, etab_v)

    nbase = pl.multiple_of(wid * NODES_PER_W, NODES_PER_W)
    pltpu.sync_copy(nodes_hbm.at[pl.ds(nbase, NODES_PER_W)], nidx_v)
    pltpu.async_copy(ntab_hbm.at[nidx_v], nrows_v, sem_m).wait()
    pltpu.sync_copy(nrows_v, node_out.at[pl.ds(nbase, NODES_PER_W)])

    @pl.when(wid == 0)
    def _():
        pltpu.sync_copy(tsteps_hbm, tidx_v)
        pltpu.async_copy(ttab_hbm.at[tidx_v], trows_v, sem_m).wait()
        pltpu.sync_copy(trows_v, time_out)

    iota16 = lax.iota(jnp.int32, 16)

    def compute_chunk(idx_ref, out_ref):
        @plsc.parallel_loop(0, GROUPS, step=1, unroll=2)
        def group(g):
            e_vec = idx_ref[pl.ds(g * 16, 16)]
            rows = g * 16 + iota16
            for d in range(D):
                tab_d = etab_v[pl.ds(d * 16, 16)]
                vals = _vperm(tab_d, e_vec)
                plsc.store_scatter(
                    out_ref, [rows, jnp.full((16,), d, jnp.int32)], vals)

    def body(p, carry):
        for par in range(2):
            c = p * 2 + par
            hbm_row = pl.multiple_of((ebase + c * CHUNK), CHUNK)
            idx_off = pl.multiple_of(ebase + (c + 2) * CHUNK, CHUNK)
            pltpu.make_async_copy(
                edges_hbm.at[pl.ds(0, CHUNK)], eidx_v.at[par], sem_i[par]
            ).wait()
            @pl.when(p > 0)
            def _():
                pltpu.make_async_copy(
                    erows_v.at[par, :, pl.ds(0, D)],
                    edge_out.at[pl.ds(0, CHUNK)], sem_o[par]
                ).wait()
            compute_chunk(eidx_v.at[par], erows_v.at[par])
            @pl.when(p < (CHUNKS // 2) - 1)
            def _():
                pltpu.async_copy(
                    edges_hbm.at[pl.ds(idx_off, CHUNK)], eidx_v.at[par],
                    sem_i[par])
            pltpu.async_copy(
                erows_v.at[par, :, pl.ds(0, D)],
                edge_out.at[pl.ds(hbm_row, CHUNK)], sem_o[par])
        return carry

    lax.fori_loop(0, CHUNKS // 2, body, 0)
    for par in range(2):
        pltpu.make_async_copy(
            erows_v.at[par, :, pl.ds(0, D)],
            edge_out.at[pl.ds(0, CHUNK)], sem_o[par]
        ).wait()


def kernel(nodes, edges, timestep, node_table, edge_table, time_table):
    tsteps = jnp.full((8,), timestep, dtype=jnp.int32)
    etab_t = jnp.pad(edge_table.T, ((0, 0), (0, 8))).reshape(D * 16)
    node_emb, edge_flat, time_rows = _sc_embed(
        nodes.astype(jnp.int32),
        edges.reshape(E).astype(jnp.int32),
        tsteps,
        node_table,
        etab_t,
        time_table,
    )
    return node_emb, edge_flat.reshape(N, N, D), time_rows[0]


# speedup vs baseline: 9.6373x; 1.6444x over previous
import functools

import jax
import jax.numpy as jnp
from jax import lax
from jax.experimental import pallas as pl
from jax.experimental.pallas import tpu as pltpu
from jax.experimental.pallas import tpu_sc as plsc

N = 1024
D = 64
NW = 32
ROWS_PER_W = N // NW
HALF = N // 2
NODES_PER_W = N // NW

_mesh = plsc.VectorSubcoreMesh(core_axis_name="c", subcore_axis_name="s")

_GATHER_DNUMS = lax.GatherDimensionNumbers(
    offset_dims=(), collapsed_slice_dims=(0,), start_index_map=(0,))


def _vperm(tab_d, e_vec):
    return lax.gather(
        tab_d, e_vec[:, None], _GATHER_DNUMS, slice_sizes=(1,),
        mode=lax.GatherScatterMode.PROMISE_IN_BOUNDS)


@functools.partial(
    pl.kernel,
    out_type=(
        jax.ShapeDtypeStruct((N, 128), jnp.float32),
        jax.ShapeDtypeStruct((N, D, N), jnp.float32),
        jax.ShapeDtypeStruct((8, 128), jnp.float32),
    ),
    mesh=_mesh,
    scratch_types=[
        pltpu.VMEM((D * 16,), jnp.float32),
        pltpu.VMEM((N,), jnp.int32),
        pltpu.VMEM((N,), jnp.int32),
        pltpu.VMEM((D, HALF), jnp.float32),
        pltpu.VMEM((D, HALF), jnp.float32),
        pltpu.VMEM((NODES_PER_W,), jnp.int32),
        pltpu.VMEM((NODES_PER_W, 128), jnp.float32),
        pltpu.VMEM((8,), jnp.int32),
        pltpu.VMEM((8, 128), jnp.float32),
        pltpu.SemaphoreType.DMA,
        pltpu.SemaphoreType.DMA,
        pltpu.SemaphoreType.DMA,
        pltpu.SemaphoreType.DMA,
        pltpu.SemaphoreType.DMA,
    ],
    compiler_params=pltpu.CompilerParams(
        use_tc_tiling_on_sc=True, needs_layout_passes=False),
)
def _sc_embed(nodes_hbm, edges_hbm, tsteps_hbm, ntab_hbm, etabt_hbm, ttab_hbm,
              node_out, edge_out, time_out,
              etab_v, eidx0_v, eidx1_v, eout0_v, eout1_v,
              nidx_v, nrows_v, tidx_v, trows_v,
              sem_i0, sem_i1, sem_o0, sem_o1, sem_m):
    wid = lax.axis_index("s") * 2 + lax.axis_index("c")
    rbase = wid * ROWS_PER_W
    eidx = (eidx0_v, eidx1_v)
    eout = (eout0_v, eout1_v)
    sem_i = (sem_i0, sem_i1)
    sem_o = (sem_o0, sem_o1)

    for rr in range(2):
        pltpu.async_copy(edges_hbm.at[rbase + rr, :], eidx[rr], sem_i[rr])
    pltpu.sync_copy(etabt_hbm, etab_v)

    nbase = pl.multiple_of(wid * NODES_PER_W, NODES_PER_W)
    pltpu.sync_copy(nodes_hbm.at[pl.ds(nbase, NODES_PER_W)], nidx_v)
    pltpu.async_copy(ntab_hbm.at[nidx_v], nrows_v, sem_m).wait()
    pltpu.sync_copy(nrows_v, node_out.at[pl.ds(nbase, NODES_PER_W), :])

    @pl.when(wid == 0)
    def _():
        pltpu.sync_copy(tsteps_hbm, tidx_v)
        pltpu.async_copy(ttab_hbm.at[tidx_v], trows_v, sem_m).wait()
        pltpu.sync_copy(trows_v, time_out)

    def pair_body(p, carry):
        for rr in range(2):
            r = p * 2 + rr
            i = rbase + r
            pltpu.make_async_copy(
                edges_hbm.at[0, :], eidx[rr], sem_i[rr]).wait()
            for h in range(2):
                if rr == 0:
                    @pl.when(p > 0)
                    def _():
                        pltpu.make_async_copy(
                            eout[h], edge_out.at[0, :, pl.ds(0, HALF)],
                            sem_o[h]).wait()
                else:
                    pltpu.make_async_copy(
                        eout[h], edge_out.at[0, :, pl.ds(0, HALF)],
                        sem_o[h]).wait()

                def group(g, c, rr=rr, h=h):
                    e_vec = eidx[rr][pl.ds(h * HALF + g * 16, 16)]
                    for d in range(D):
                        tab_d = etab_v[pl.ds(d * 16, 16)]
                        eout[h][d, pl.ds(g * 16, 16)] = _vperm(tab_d, e_vec)
                    return c
                lax.fori_loop(0, HALF // 16, group, 0)
                pltpu.async_copy(
                    eout[h], edge_out.at[i, :, pl.ds(h * HALF, HALF)],
                    sem_o[h])
            @pl.when(p < (ROWS_PER_W // 2) - 1)
            def _():
                pltpu.async_copy(
                    edges_hbm.at[i + 2, :], eidx[rr], sem_i[rr])
        return carry

    lax.fori_loop(0, ROWS_PER_W // 2, pair_body, 0)
    for h in range(2):
        pltpu.make_async_copy(
            eout[h], edge_out.at[0, :, pl.ds(0, HALF)], sem_o[h]).wait()


def kernel(nodes, edges, timestep, node_table, edge_table, time_table):
    tsteps = jnp.full((8,), timestep, dtype=jnp.int32)
    etab_t = jnp.pad(edge_table.T, ((0, 0), (0, 8))).reshape(D * 16)
    node_rows, edge_idj, time_rows = _sc_embed(
        nodes.astype(jnp.int32),
        edges.astype(jnp.int32),
        tsteps,
        jnp.pad(node_table, ((0, 0), (0, 64))),
        etab_t,
        jnp.pad(time_table, ((0, 0), (0, 64))),
    )
    return (node_rows[:, :D],
            jnp.transpose(edge_idj, (0, 2, 1)),
            time_rows[0, :D])


# speedup vs baseline: 33.3516x; 3.4607x over previous
import functools

import jax
import jax.numpy as jnp
from jax import lax
from jax.experimental import pallas as pl
from jax.experimental.pallas import tpu as pltpu
from jax.experimental.pallas import tpu_sc as plsc

N = 1024
D = 64
NW = 32
ROWS_PER_W = N // NW
HALF = N // 2
NODES_PER_W = N // NW

_mesh = plsc.VectorSubcoreMesh(core_axis_name="c", subcore_axis_name="s")

_GATHER_DNUMS = lax.GatherDimensionNumbers(
    offset_dims=(), collapsed_slice_dims=(0,), start_index_map=(0,))


def _vperm(tab_d, e_vec):
    return lax.gather(
        tab_d, e_vec[:, None], _GATHER_DNUMS, slice_sizes=(1,),
        mode=lax.GatherScatterMode.PROMISE_IN_BOUNDS)


@functools.partial(
    pl.kernel,
    out_type=(
        jax.ShapeDtypeStruct((N, 128), jnp.float32),
        jax.ShapeDtypeStruct((N, D, N), jnp.float32),
        jax.ShapeDtypeStruct((8, 128), jnp.float32),
    ),
    mesh=_mesh,
    scratch_types=[
        pltpu.VMEM((D * 16,), jnp.float32),
        pltpu.VMEM((N,), jnp.int32),
        pltpu.VMEM((N,), jnp.int32),
        pltpu.VMEM((D, HALF), jnp.float32),
        pltpu.VMEM((D, HALF), jnp.float32),
        pltpu.VMEM((NODES_PER_W,), jnp.int32),
        pltpu.VMEM((NODES_PER_W, 128), jnp.float32),
        pltpu.VMEM((8,), jnp.int32),
        pltpu.VMEM((8, 128), jnp.float32),
        pltpu.SemaphoreType.DMA,
        pltpu.SemaphoreType.DMA,
        pltpu.SemaphoreType.DMA,
        pltpu.SemaphoreType.DMA,
        pltpu.SemaphoreType.DMA,
    ],
    compiler_params=pltpu.CompilerParams(
        use_tc_tiling_on_sc=True, needs_layout_passes=False),
)
def _sc_embed(nodes_hbm, edges_hbm, tsteps_hbm, ntab_hbm, etabt_hbm, ttab_hbm,
              node_out, edge_out, time_out,
              etab_v, eidx0_v, eidx1_v, eout0_v, eout1_v,
              nidx_v, nrows_v, tidx_v, trows_v,
              sem_i0, sem_i1, sem_o0, sem_o1, sem_m):
    wid = lax.axis_index("s") * 2 + lax.axis_index("c")
    rbase = wid * ROWS_PER_W
    eidx = (eidx0_v, eidx1_v)
    eout = (eout0_v, eout1_v)
    sem_i = (sem_i0, sem_i1)
    sem_o = (sem_o0, sem_o1)

    for rr in range(2):
        pltpu.async_copy(edges_hbm.at[rbase + rr, :], eidx[rr], sem_i[rr])
    pltpu.sync_copy(etabt_hbm, etab_v)

    nbase = pl.multiple_of(wid * NODES_PER_W, NODES_PER_W)
    pltpu.sync_copy(nodes_hbm.at[pl.ds(nbase, NODES_PER_W)], nidx_v)
    pltpu.async_copy(ntab_hbm.at[nidx_v], nrows_v, sem_m).wait()
    pltpu.sync_copy(nrows_v, node_out.at[pl.ds(nbase, NODES_PER_W), :])

    @pl.when(wid == 0)
    def _():
        pltpu.sync_copy(tsteps_hbm, tidx_v)
        pltpu.async_copy(ttab_hbm.at[tidx_v], trows_v, sem_m).wait()
        pltpu.sync_copy(trows_v, time_out)

    def pair_body(p, carry):
        for rr in range(2):
            r = p * 2 + rr
            i = rbase + r
            pltpu.make_async_copy(
                edges_hbm.at[0, :], eidx[rr], sem_i[rr]).wait()
            for h in range(2):
                if rr == 0:
                    @pl.when(p > 0)
                    def _():
                        pltpu.make_async_copy(
                            eout[h], edge_out.at[0, :, pl.ds(0, HALF)],
                            sem_o[h]).wait()
                else:
                    pltpu.make_async_copy(
                        eout[h], edge_out.at[0, :, pl.ds(0, HALF)],
                        sem_o[h]).wait()

                def make_group(rr, h):
                    def group(g):
                        e_vec = eidx[rr][pl.ds(h * HALF + g * 16, 16)]
                        for d in range(D):
                            tab_d = etab_v[pl.ds(d * 16, 16)]
                            eout[h][d, pl.ds(g * 16, 16)] = _vperm(tab_d, e_vec)
                    return group
                plsc.parallel_loop(0, HALF // 16, step=1, unroll=2)(
                    make_group(rr, h))
                pltpu.async_copy(
                    eout[h], edge_out.at[i, :, pl.ds(h * HALF, HALF)],
                    sem_o[h])
            @pl.when(p < (ROWS_PER_W // 2) - 1)
            def _():
                pltpu.async_copy(
                    edges_hbm.at[i + 2, :], eidx[rr], sem_i[rr])
        return carry

    lax.fori_loop(0, ROWS_PER_W // 2, pair_body, 0)
    for h in range(2):
        pltpu.make_async_copy(
            eout[h], edge_out.at[0, :, pl.ds(0, HALF)], sem_o[h]).wait()


def kernel(nodes, edges, timestep, node_table, edge_table, time_table):
    tsteps = jnp.full((8,), timestep, dtype=jnp.int32)
    etab_t = jnp.pad(edge_table.T, ((0, 0), (0, 8))).reshape(D * 16)
    node_rows, edge_idj, time_rows = _sc_embed(
        nodes.astype(jnp.int32),
        edges.astype(jnp.int32),
        tsteps,
        jnp.pad(node_table, ((0, 0), (0, 64))),
        etab_t,
        jnp.pad(time_table, ((0, 0), (0, 64))),
    )
    return (node_rows[:, :D],
            jnp.transpose(edge_idj, (0, 2, 1)),
            time_rows[0, :D])
